# baseline (speedup 1.0000x reference)
import functools

import jax
import jax.numpy as jnp
from jax import lax
from jax.experimental import pallas as pl
from jax.experimental.pallas import tpu as pltpu
from jax.experimental.pallas import tpu_sc as plsc

BATCH = 16384
D = 128

_NPIPE = 1
_PB = BATCH // _NPIPE


_NC, _NS = 2, 16
_NW = _NC * _NS
_BPW = _PB // _NW
_CH = 128
_NCHUNK = _BPW // _CH

@functools.lru_cache(maxsize=1)
def _make_sc_gather():
    mesh = plsc.VectorSubcoreMesh(
        core_axis_name="c", subcore_axis_name="s",
        num_cores=_NC, num_subcores=_NS)

    @functools.partial(
        pl.kernel,
        mesh=mesh,
        out_type=jax.ShapeDtypeStruct((_PB, 2 * D), jnp.float32),
        scratch_types=[
            pltpu.VMEM((_NCHUNK, _CH), jnp.int32),
            pltpu.VMEM((_NCHUNK, _CH), jnp.int32),
            pltpu.VMEM((_CH, D), jnp.float32),
            pltpu.VMEM((_CH, D), jnp.float32),
            pltpu.SemaphoreType.DMA,
            pltpu.SemaphoreType.DMA,
            pltpu.SemaphoreType.DMA,
            pltpu.SemaphoreType.DMA,
        ],
    )
    def _sc_gather(users_hbm, movies_hbm, eu_hbm, em_hbm, x_out,
                   uidx_v, midx_v, rows0_v, rows1_v, gs0, gs1, ss0, ss1):
        wid = lax.axis_index("s") * _NC + lax.axis_index("c")
        base = wid * _BPW
        pltpu.sync_copy(users_hbm.at[pl.ds(wid * _NCHUNK, _NCHUNK)], uidx_v)
        pltpu.sync_copy(movies_hbm.at[pl.ds(wid * _NCHUNK, _NCHUNK)], midx_v)
        chunks = (
            [(eu_hbm, uidx_v.at[j], base + j * _CH, 0)
             for j in range(_NCHUNK)]
            + [(em_hbm, midx_v.at[j], base + j * _CH, D)
               for j in range(_NCHUNK)]
        )
        bufs = (rows0_v, rows1_v)
        gsems = (gs0, gs1)
        ssems = (ss0, ss1)
        tbl0, idx0, _, _ = chunks[0]
        pend_g = pltpu.async_copy(tbl0.at[idx0], bufs[0], gsems[0])
        pend_s = [None, None]
        for k, (_, _, roff, coff) in enumerate(chunks):
            pend_g.wait()
            if k + 1 < len(chunks):
                b = (k + 1) % 2
                if pend_s[b] is not None:
                    pend_s[b].wait()
                    pend_s[b] = None
                tbl, idx, _, _ = chunks[k + 1]
                pend_g = pltpu.async_copy(tbl.at[idx], bufs[b], gsems[b])
            pend_s[k % 2] = pltpu.async_copy(
                bufs[k % 2], x_out.at[pl.ds(roff, _CH), pl.ds(coff, D)],
                ssems[k % 2])
        for s in pend_s:
            if s is not None:
                s.wait()

    return _sc_gather


_BM = 2048


def _mlp_body(x_ref, w1_ref, b1_ref, w2_ref, b2_ref, wf_ref, bf_ref, out_ref):
    xin = x_ref[...].astype(jnp.bfloat16)
    x = jnp.dot(xin, w1_ref[...], preferred_element_type=jnp.float32)
    x = jax.nn.relu(x + b1_ref[...]).astype(jnp.bfloat16)
    x = jnp.dot(x, w2_ref[...], preferred_element_type=jnp.float32)
    x = jax.nn.relu(x + b2_ref[...]).astype(jnp.bfloat16)
    x = jnp.dot(x, wf_ref[...], preferred_element_type=jnp.float32)
    out_ref[...] = jax.nn.sigmoid(x + bf_ref[...])


def _mlp_chunk0(x, w1, b1, w2, b2, wf, bf):
    h1, h2 = w2.shape
    grid = (_PB // _BM,)
    return pl.pallas_call(
        _mlp_body,
        grid=grid,
        in_specs=[
            pl.BlockSpec((_BM, 2 * D), lambda i: (i, 0)),
            pl.BlockSpec((2 * D, h1), lambda i: (0, 0)),
            pl.BlockSpec((1, h1), lambda i: (0, 0)),
            pl.BlockSpec((h1, h2), lambda i: (0, 0)),
            pl.BlockSpec((1, h2), lambda i: (0, 0)),
            pl.BlockSpec((h2, 1), lambda i: (0, 0)),
            pl.BlockSpec((1, 1), lambda i: (0, 0)),
        ],
        out_specs=pl.BlockSpec((_BM, 1), lambda i: (i, 0)),
        out_shape=jax.ShapeDtypeStruct((BATCH, 1), jnp.float32),
    )(x, w1, b1, w2, b2, wf, bf)


def _mlp_body_acc(x_ref, w1_ref, b1_ref, w2_ref, b2_ref, wf_ref, bf_ref,
                  prev_ref, out_ref):
    del prev_ref
    _mlp_body(x_ref, w1_ref, b1_ref, w2_ref, b2_ref, wf_ref, bf_ref, out_ref)


def _mlp_chunk1(x, w1, b1, w2, b2, wf, bf, prev):
    h1, h2 = w2.shape
    nb = _PB // _BM
    grid = (nb,)
    return pl.pallas_call(
        _mlp_body_acc,
        grid=grid,
        in_specs=[
            pl.BlockSpec((_BM, 2 * D), lambda i: (i, 0)),
            pl.BlockSpec((2 * D, h1), lambda i: (0, 0)),
            pl.BlockSpec((1, h1), lambda i: (0, 0)),
            pl.BlockSpec((h1, h2), lambda i: (0, 0)),
            pl.BlockSpec((1, h2), lambda i: (0, 0)),
            pl.BlockSpec((h2, 1), lambda i: (0, 0)),
            pl.BlockSpec((1, 1), lambda i: (0, 0)),
            pl.BlockSpec(memory_space=pl.ANY),
        ],
        out_specs=pl.BlockSpec((_BM, 1), lambda i: (nb + i, 0)),
        out_shape=jax.ShapeDtypeStruct((BATCH, 1), jnp.float32),
        input_output_aliases={7: 0},
    )(x, w1, b1, w2, b2, wf, bf, prev)


def kernel(users, movies, Eu, Em, W1, b1, W2, b2, Wf, bf):
    h1, h2 = W2.shape
    u2 = users.astype(jnp.int32).reshape(_NPIPE, _PB // _CH, _CH)
    m2 = movies.astype(jnp.int32).reshape(_NPIPE, _PB // _CH, _CH)
    w1c = W1.astype(jnp.bfloat16)
    b1r = b1.reshape(1, h1)
    w2c = W2.astype(jnp.bfloat16)
    b2r = b2.reshape(1, h2)
    wfc = Wf.astype(jnp.bfloat16)
    bfr = bf.reshape(1, 1)
    sc = _make_sc_gather()
    if _NPIPE == 1:
        x0 = sc(u2[0], m2[0], Eu, Em)
        return _mlp_chunk0(x0, w1c, b1r, w2c, b2r, wfc, bfr)
    x0 = sc(u2[0], m2[0], Eu, Em)
    x1 = sc(u2[1], m2[1], Eu, Em)
    out = _mlp_chunk0(x0, w1c, b1r, w2c, b2r, wfc, bfr)
    return _mlp_chunk1(x1, w1c, b1r, w2c, b2r, wfc, bfr, out)


# speedup vs baseline: 1.0625x; 1.0625x over previous
import functools

import jax
import jax.numpy as jnp
from jax import lax
from jax.experimental import pallas as pl
from jax.experimental.pallas import tpu as pltpu
from jax.experimental.pallas import tpu_sc as plsc

BATCH = 16384
D = 128

_NPIPE = 2
_PB = BATCH // _NPIPE


_NC, _NS = 2, 16
_NW = _NC * _NS
_BPW = _PB // _NW
_CH = 128
_NCHUNK = _BPW // _CH

@functools.lru_cache(maxsize=1)
def _make_sc_gather():
    mesh = plsc.VectorSubcoreMesh(
        core_axis_name="c", subcore_axis_name="s",
        num_cores=_NC, num_subcores=_NS)

    @functools.partial(
        pl.kernel,
        mesh=mesh,
        out_type=jax.ShapeDtypeStruct((_PB, 2 * D), jnp.float32),
        scratch_types=[
            pltpu.VMEM((_NCHUNK, _CH), jnp.int32),
            pltpu.VMEM((_NCHUNK, _CH), jnp.int32),
            pltpu.VMEM((_CH, D), jnp.float32),
            pltpu.VMEM((_CH, D), jnp.float32),
            pltpu.SemaphoreType.DMA,
            pltpu.SemaphoreType.DMA,
            pltpu.SemaphoreType.DMA,
            pltpu.SemaphoreType.DMA,
        ],
    )
    def _sc_gather(users_hbm, movies_hbm, eu_hbm, em_hbm, x_out,
                   uidx_v, midx_v, rows0_v, rows1_v, gs0, gs1, ss0, ss1):
        wid = lax.axis_index("s") * _NC + lax.axis_index("c")
        base = wid * _BPW
        pltpu.sync_copy(users_hbm.at[pl.ds(wid * _NCHUNK, _NCHUNK)], uidx_v)
        pltpu.sync_copy(movies_hbm.at[pl.ds(wid * _NCHUNK, _NCHUNK)], midx_v)
        chunks = (
            [(eu_hbm, uidx_v.at[j], base + j * _CH, 0)
             for j in range(_NCHUNK)]
            + [(em_hbm, midx_v.at[j], base + j * _CH, D)
               for j in range(_NCHUNK)]
        )
        bufs = (rows0_v, rows1_v)
        gsems = (gs0, gs1)
        ssems = (ss0, ss1)
        tbl0, idx0, _, _ = chunks[0]
        pend_g = pltpu.async_copy(tbl0.at[idx0], bufs[0], gsems[0])
        pend_s = [None, None]
        for k, (_, _, roff, coff) in enumerate(chunks):
            pend_g.wait()
            if k + 1 < len(chunks):
                b = (k + 1) % 2
                if pend_s[b] is not None:
                    pend_s[b].wait()
                    pend_s[b] = None
                tbl, idx, _, _ = chunks[k + 1]
                pend_g = pltpu.async_copy(tbl.at[idx], bufs[b], gsems[b])
            pend_s[k % 2] = pltpu.async_copy(
                bufs[k % 2], x_out.at[pl.ds(roff, _CH), pl.ds(coff, D)],
                ssems[k % 2])
        for s in pend_s:
            if s is not None:
                s.wait()

    return _sc_gather


_BM = 2048


def _mlp_body(x_ref, w1_ref, b1_ref, w2_ref, b2_ref, wf_ref, bf_ref, out_ref):
    xin = x_ref[...].astype(jnp.bfloat16)
    x = jnp.dot(xin, w1_ref[...], preferred_element_type=jnp.float32)
    x = jax.nn.relu(x + b1_ref[...]).astype(jnp.bfloat16)
    x = jnp.dot(x, w2_ref[...], preferred_element_type=jnp.float32)
    x = jax.nn.relu(x + b2_ref[...]).astype(jnp.bfloat16)
    x = jnp.dot(x, wf_ref[...], preferred_element_type=jnp.float32)
    out_ref[...] = jax.nn.sigmoid(x + bf_ref[...])


def _mlp_chunk0(x, w1, b1, w2, b2, wf, bf):
    h1, h2 = w2.shape
    grid = (_PB // _BM,)
    return pl.pallas_call(
        _mlp_body,
        grid=grid,
        in_specs=[
            pl.BlockSpec((_BM, 2 * D), lambda i: (i, 0)),
            pl.BlockSpec((2 * D, h1), lambda i: (0, 0)),
            pl.BlockSpec((1, h1), lambda i: (0, 0)),
            pl.BlockSpec((h1, h2), lambda i: (0, 0)),
            pl.BlockSpec((1, h2), lambda i: (0, 0)),
            pl.BlockSpec((h2, 1), lambda i: (0, 0)),
            pl.BlockSpec((1, 1), lambda i: (0, 0)),
        ],
        out_specs=pl.BlockSpec((_BM, 1), lambda i: (i, 0)),
        out_shape=jax.ShapeDtypeStruct((BATCH, 1), jnp.float32),
    )(x, w1, b1, w2, b2, wf, bf)


def _mlp_body_acc(x_ref, w1_ref, b1_ref, w2_ref, b2_ref, wf_ref, bf_ref,
                  prev_ref, out_ref):
    del prev_ref
    _mlp_body(x_ref, w1_ref, b1_ref, w2_ref, b2_ref, wf_ref, bf_ref, out_ref)


def _mlp_chunk1(x, w1, b1, w2, b2, wf, bf, prev):
    h1, h2 = w2.shape
    nb = _PB // _BM
    grid = (nb,)
    return pl.pallas_call(
        _mlp_body_acc,
        grid=grid,
        in_specs=[
            pl.BlockSpec((_BM, 2 * D), lambda i: (i, 0)),
            pl.BlockSpec((2 * D, h1), lambda i: (0, 0)),
            pl.BlockSpec((1, h1), lambda i: (0, 0)),
            pl.BlockSpec((h1, h2), lambda i: (0, 0)),
            pl.BlockSpec((1, h2), lambda i: (0, 0)),
            pl.BlockSpec((h2, 1), lambda i: (0, 0)),
            pl.BlockSpec((1, 1), lambda i: (0, 0)),
            pl.BlockSpec(memory_space=pl.ANY),
        ],
        out_specs=pl.BlockSpec((_BM, 1), lambda i: (nb + i, 0)),
        out_shape=jax.ShapeDtypeStruct((BATCH, 1), jnp.float32),
        input_output_aliases={7: 0},
    )(x, w1, b1, w2, b2, wf, bf, prev)


def kernel(users, movies, Eu, Em, W1, b1, W2, b2, Wf, bf):
    h1, h2 = W2.shape
    u2 = users.astype(jnp.int32).reshape(_NPIPE, _PB // _CH, _CH)
    m2 = movies.astype(jnp.int32).reshape(_NPIPE, _PB // _CH, _CH)
    w1c = W1.astype(jnp.bfloat16)
    b1r = b1.reshape(1, h1)
    w2c = W2.astype(jnp.bfloat16)
    b2r = b2.reshape(1, h2)
    wfc = Wf.astype(jnp.bfloat16)
    bfr = bf.reshape(1, 1)
    sc = _make_sc_gather()
    x0 = sc(u2[0], m2[0], Eu, Em)
    x1 = sc(u2[1], m2[1], Eu, Em)
    out = _mlp_chunk0(x0, w1c, b1r, w2c, b2r, wfc, bfr)
    return _mlp_chunk1(x1, w1c, b1r, w2c, b2r, wfc, bfr, out)
